# baseline (speedup 1.0000x reference)
import jax
import jax.numpy as jnp
from jax.experimental import pallas as pl
from jax.experimental.pallas import tpu as pltpu

_B = 256
_D = 256
_K = 1024
_DH = _D // 2


def _som_kernel(x_ref, w_hbm, loc_ref, bmu_ref, loss_ref, w_v, sem):
    cp0 = pltpu.make_async_copy(w_hbm.at[pl.ds(0, _DH), :],
                                w_v.at[pl.ds(0, _DH), :], sem.at[0])
    cp1 = pltpu.make_async_copy(w_hbm.at[pl.ds(_DH, _DH), :],
                                w_v.at[pl.ds(_DH, _DH), :], sem.at[1])
    cp0.start()
    cp1.start()
    x = x_ref[...] + 1e-6
    xsq = jnp.sum(x * x, axis=1)
    cp0.wait()
    w0 = w_v[pl.ds(0, _DH), :]
    xw = jax.lax.dot_general(
        x[:, :_DH], w0, (((1,), (0,)), ((), ())),
        preferred_element_type=jnp.float32,
        precision=jax.lax.Precision.HIGHEST,
    )
    wsq0 = jnp.sum(w0 * w0, axis=0, keepdims=True)
    cp1.wait()
    w1 = w_v[pl.ds(_DH, _DH), :]
    xw = xw + jax.lax.dot_general(
        x[:, _DH:], w1, (((1,), (0,)), ((), ())),
        preferred_element_type=jnp.float32,
        precision=jax.lax.Precision.HIGHEST,
    )
    wsq_half = 0.5 * (wsq0 + jnp.sum(w1 * w1, axis=0, keepdims=True))
    s = wsq_half - xw
    min_s = jnp.min(s, axis=1)
    idx = jnp.argmin(s, axis=1)
    d2min = jnp.maximum(xsq + 2.0 * min_s, 0.0)
    loss_ref[...] = jnp.reshape(
        jnp.sum(jnp.sqrt(d2min)) / jnp.float32(_B), (1, 1))
    onehot = (jax.lax.broadcasted_iota(jnp.int32, (_B, _K), 1)
              == idx[:, None]).astype(jnp.bfloat16)
    bmu_ref[...] = jax.lax.dot_general(
        onehot, loc_ref[...].astype(jnp.bfloat16), (((1,), (0,)), ((), ())),
        preferred_element_type=jnp.float32,
    )


def kernel(input, weight, locations):
    bmu, loss = pl.pallas_call(
        _som_kernel,
        in_specs=[
            pl.BlockSpec(memory_space=pltpu.VMEM),
            pl.BlockSpec(memory_space=pltpu.HBM),
            pl.BlockSpec(memory_space=pltpu.VMEM),
        ],
        out_specs=(
            pl.BlockSpec(memory_space=pltpu.VMEM),
            pl.BlockSpec(memory_space=pltpu.VMEM),
        ),
        out_shape=(
            jax.ShapeDtypeStruct((_B, 2), jnp.float32),
            jax.ShapeDtypeStruct((1, 1), jnp.float32),
        ),
        scratch_shapes=[
            pltpu.VMEM((_D, _K), jnp.float32),
            pltpu.SemaphoreType.DMA((2,)),
        ],
    )(input, weight, locations)
    return bmu.reshape(_B, 1, 2), loss.reshape(())


# speedup vs baseline: 1.2005x; 1.2005x over previous
import jax
import jax.numpy as jnp
from jax.experimental import pallas as pl

_B = 256
_D = 256
_K = 1024


def _som_kernel(x_ref, w_ref, loc_ref, bmu_ref, loss_ref):
    x = x_ref[...] + 1e-6
    w = w_ref[...]
    wsq_half = 0.5 * jnp.sum(w * w, axis=0, keepdims=True)
    xw = jax.lax.dot_general(
        x, w, (((1,), (0,)), ((), ())),
        preferred_element_type=jnp.float32,
        precision=jax.lax.Precision.HIGHEST,
    )
    s = wsq_half - xw
    min_s = jnp.min(s, axis=1)
    idx = jnp.argmin(s, axis=1)
    xsq = jnp.sum(x * x, axis=1)
    d2min = jnp.maximum(xsq + 2.0 * min_s, 0.0)
    loss_ref[...] = jnp.reshape(
        jnp.sum(jnp.sqrt(d2min)) / jnp.float32(_B), (1, 1))
    onehot = (jax.lax.broadcasted_iota(jnp.int32, (_B, _K), 1)
              == idx[:, None]).astype(jnp.bfloat16)
    bmu_ref[...] = jax.lax.dot_general(
        onehot, loc_ref[...].astype(jnp.bfloat16), (((1,), (0,)), ((), ())),
        preferred_element_type=jnp.float32,
    )


def kernel(input, weight, locations):
    bmu, loss = pl.pallas_call(
        _som_kernel,
        out_shape=(
            jax.ShapeDtypeStruct((_B, 2), jnp.float32),
            jax.ShapeDtypeStruct((1, 1), jnp.float32),
        ),
    )(input, weight, locations)
    return bmu.reshape(_B, 1, 2), loss.reshape(())
